# baseline (speedup 1.0000x reference)
import functools

import jax
import jax.numpy as jnp
from jax import lax
from jax.experimental import pallas as pl
from jax.experimental.pallas import tpu as pltpu
from jax.experimental.pallas import tpu_sc as plsc

_B = 8
_T = 512
_E = 512
_H = 512
_G = 4 * _H
_CH = 64
_NK = _T // _CH


def _sc_gather(tokens, fidx, emb):
    info = plsc.get_sparse_core_info()
    nw = info.num_cores * info.num_subcores
    rows = _B * _T
    rpw = rows // nw
    mesh = plsc.VectorSubcoreMesh(core_axis_name="c", subcore_axis_name="s")

    @functools.partial(
        pl.kernel,
        mesh=mesh,
        out_type=jax.ShapeDtypeStruct((rows, _E), jnp.float32),
        scratch_types=[
            pltpu.VMEM((rpw,), jnp.int32),
            pltpu.VMEM((rpw,), jnp.int32),
            pltpu.VMEM((rpw, _E), jnp.float32),
            pltpu.SemaphoreType.DMA,
            pltpu.SemaphoreType.DMA,
        ],
    )
    def gather_kernel(tok_hbm, fidx_hbm, emb_hbm, x_hbm,
                      fidx_v, tid_v, rows_v, sem1, sem2):
        wid = lax.axis_index("s") * info.num_cores + lax.axis_index("c")
        base = wid * rpw
        pltpu.sync_copy(fidx_hbm.at[pl.ds(base, rpw)], fidx_v)
        pltpu.async_copy(tok_hbm.at[fidx_v], tid_v, sem1).wait()
        pltpu.async_copy(emb_hbm.at[tid_v], rows_v, sem2).wait()
        pltpu.sync_copy(rows_v, x_hbm.at[pl.ds(base, rpw)])

    return gather_kernel(tokens, fidx, emb)


def _gates1(x, wt, bsum):
    rows, k = x.shape
    blk = 512
    nblk = rows // blk

    def body(x_ref, w_ref, b_ref, gf_ref, gb_ref):
        xb = x_ref[...].astype(jnp.bfloat16)
        gf_ref[...] = (jnp.dot(xb, w_ref[0], preferred_element_type=jnp.float32)
                       + b_ref[0:1, :])
        gb_ref[...] = (jnp.dot(xb, w_ref[1], preferred_element_type=jnp.float32)
                       + b_ref[1:2, :])

    return pl.pallas_call(
        body,
        grid=(nblk,),
        in_specs=[
            pl.BlockSpec((blk, k), lambda i: (i, 0)),
            pl.BlockSpec((2, k, _G), lambda i: (0, 0, 0)),
            pl.BlockSpec((2, _G), lambda i: (0, 0)),
        ],
        out_specs=[
            pl.BlockSpec((blk, _G), lambda i: (i, 0)),
            pl.BlockSpec((blk, _G), lambda i: (i, 0)),
        ],
        out_shape=[
            jax.ShapeDtypeStruct((rows, _G), jnp.float32),
            jax.ShapeDtypeStruct((rows, _G), jnp.float32),
        ],
    )(x, wt, bsum)


def _gates2(xa, xb, wta, wtb, bsum):
    rows, k = xa.shape
    blk = 512
    nblk = rows // blk

    def body(xa_ref, xb_ref, wa_ref, wb_ref, b_ref, gf_ref, gb_ref):
        a = xa_ref[...].astype(jnp.bfloat16)
        b = xb_ref[...].astype(jnp.bfloat16)
        gf_ref[...] = (jnp.dot(a, wa_ref[0], preferred_element_type=jnp.float32)
                       + jnp.dot(b, wb_ref[0], preferred_element_type=jnp.float32)
                       + b_ref[0:1, :])
        gb_ref[...] = (jnp.dot(a, wa_ref[1], preferred_element_type=jnp.float32)
                       + jnp.dot(b, wb_ref[1], preferred_element_type=jnp.float32)
                       + b_ref[1:2, :])

    return pl.pallas_call(
        body,
        grid=(nblk,),
        in_specs=[
            pl.BlockSpec((blk, k), lambda i: (i, 0)),
            pl.BlockSpec((blk, k), lambda i: (i, 0)),
            pl.BlockSpec((2, k, _G), lambda i: (0, 0, 0)),
            pl.BlockSpec((2, k, _G), lambda i: (0, 0, 0)),
            pl.BlockSpec((2, _G), lambda i: (0, 0)),
        ],
        out_specs=[
            pl.BlockSpec((blk, _G), lambda i: (i, 0)),
            pl.BlockSpec((blk, _G), lambda i: (i, 0)),
        ],
        out_shape=[
            jax.ShapeDtypeStruct((rows, _G), jnp.float32),
            jax.ShapeDtypeStruct((rows, _G), jnp.float32),
        ],
    )(xa, xb, wta, wtb, bsum)


def _lstm_cell(g, h, c, m):
    i = jax.nn.sigmoid(g[:, 0:_H])
    f = jax.nn.sigmoid(g[:, _H:2 * _H])
    u = jnp.tanh(g[:, 2 * _H:3 * _H])
    o = jax.nn.sigmoid(g[:, 3 * _H:])
    cn = f * c + i * u
    hn = o * jnp.tanh(cn)
    sel = m > 0.5
    return jnp.where(sel, hn, h), jnp.where(sel, cn, c)


def _recur(gf, gb, mask, wf, wb, want_hs):
    rows = _CH * _B

    def body(gf_ref, gb_ref, mf_ref, mb_ref, wf_ref, wb_ref, *rest):
        outs = rest[:-4]
        hf, cf, hb, cb = rest[-4:]

        @pl.when(pl.program_id(0) == 0)
        def _init():
            hf[...] = jnp.zeros_like(hf)
            cf[...] = jnp.zeros_like(cf)
            hb[...] = jnp.zeros_like(hb)
            cb[...] = jnp.zeros_like(cb)

        def step(s, _):
            rf = pl.ds(s * _B, _B)
            g = gf_ref[rf, :] + jnp.dot(hf[...].astype(jnp.bfloat16),
                                        wf_ref[...],
                                        preferred_element_type=jnp.float32)
            hn, cn = _lstm_cell(g, hf[...], cf[...], mf_ref[rf, :])
            hf[...] = hn
            cf[...] = cn
            if want_hs:
                outs[0][rf, :] = hn

            rb = pl.ds(((_CH - 1) - s) * _B, _B)
            g2 = gb_ref[rb, :] + jnp.dot(hb[...].astype(jnp.bfloat16),
                                         wb_ref[...],
                                         preferred_element_type=jnp.float32)
            hn2, cn2 = _lstm_cell(g2, hb[...], cb[...], mb_ref[rb, :])
            hb[...] = hn2
            cb[...] = cn2
            if want_hs:
                outs[1][rb, :] = hn2
            return 0

        lax.fori_loop(0, _CH, step, 0, unroll=8)
        if not want_hs:
            outs[0][:, 0:_H] = hf[...]
            outs[0][:, _H:] = hb[...]

    fwd_map = lambda k: (k, 0)
    bwd_map = lambda k: (_NK - 1 - k, 0)
    in_specs = [
        pl.BlockSpec((rows, _G), fwd_map),
        pl.BlockSpec((rows, _G), bwd_map),
        pl.BlockSpec((rows, 1), fwd_map),
        pl.BlockSpec((rows, 1), bwd_map),
        pl.BlockSpec((_H, _G), lambda k: (0, 0)),
        pl.BlockSpec((_H, _G), lambda k: (0, 0)),
    ]
    wf = wf.astype(jnp.bfloat16)
    wb = wb.astype(jnp.bfloat16)
    if want_hs:
        out_specs = [pl.BlockSpec((rows, _H), fwd_map),
                     pl.BlockSpec((rows, _H), bwd_map)]
        out_shape = [jax.ShapeDtypeStruct((_T * _B, _H), jnp.float32),
                     jax.ShapeDtypeStruct((_T * _B, _H), jnp.float32)]
    else:
        out_specs = [pl.BlockSpec((_B, 2 * _H), lambda k: (0, 0))]
        out_shape = [jax.ShapeDtypeStruct((_B, 2 * _H), jnp.float32)]

    return pl.pallas_call(
        body,
        grid=(_NK,),
        in_specs=in_specs,
        out_specs=out_specs,
        out_shape=out_shape,
        scratch_shapes=[
            pltpu.VMEM((_B, _H), jnp.float32),
            pltpu.VMEM((_B, _H), jnp.float32),
            pltpu.VMEM((_B, _H), jnp.float32),
            pltpu.VMEM((_B, _H), jnp.float32),
        ],
    )(gf, gb, mask, mask, wf, wb)


def kernel(tokens, cu_seqlens, emb, Wih0, Whh0, bih0, bhh0,
           Wih1, Whh1, bih1, bhh1):
    total = tokens.shape[0]
    cu = cu_seqlens.astype(jnp.int32)
    lengths = cu[1:] - cu[:-1]
    t = jnp.arange(_T, dtype=jnp.int32)
    fidx = jnp.clip(cu[:-1][None, :] + t[:, None], 0, total - 1)
    fidx = fidx.reshape(-1)
    mask = (t[:, None] < lengths[None, :]).astype(jnp.float32).reshape(-1, 1)

    x = _sc_gather(tokens.astype(jnp.int32), fidx, emb)

    w0t = jnp.transpose(Wih0, (0, 2, 1)).astype(jnp.bfloat16)
    g0f, g0b = _gates1(x, w0t, bih0 + bhh0)
    whh0t = jnp.transpose(Whh0, (0, 2, 1))
    hsf, hsb = _recur(g0f, g0b, mask, whh0t[0], whh0t[1], want_hs=True)

    w1t = jnp.transpose(Wih1, (0, 2, 1)).astype(jnp.bfloat16)
    g1f, g1b = _gates2(hsf, hsb, w1t[:, :_H, :], w1t[:, _H:, :], bih1 + bhh1)
    whh1t = jnp.transpose(Whh1, (0, 2, 1))
    (h_final,) = _recur(g1f, g1b, mask, whh1t[0], whh1t[1], want_hs=False)
    return h_final


# speedup vs baseline: 1.0318x; 1.0318x over previous
import functools

import jax
import jax.numpy as jnp
from jax import lax
from jax.experimental import pallas as pl
from jax.experimental.pallas import tpu as pltpu
from jax.experimental.pallas import tpu_sc as plsc

_B = 8
_T = 512
_E = 512
_H = 512
_G = 4 * _H
_CH = 64
_NK = _T // _CH


def _sc_gather(tokens, dst, emb):
    info = plsc.get_sparse_core_info()
    nw = info.num_cores * info.num_subcores
    total = tokens.shape[0]
    rpw = total // nw
    rows = _B * _T
    mesh = plsc.VectorSubcoreMesh(core_axis_name="c", subcore_axis_name="s")

    @functools.partial(
        pl.kernel,
        mesh=mesh,
        out_type=jax.ShapeDtypeStruct((rows, _E), jnp.float32),
        scratch_types=[
            pltpu.VMEM((rpw,), jnp.int32),
            pltpu.VMEM((rpw,), jnp.int32),
            pltpu.VMEM((rpw, _E), jnp.float32),
            pltpu.SemaphoreType.DMA,
            pltpu.SemaphoreType.DMA,
        ],
    )
    def gather_kernel(tok_hbm, dst_hbm, emb_hbm, x_hbm,
                      tid_v, dst_v, rows_v, sem1, sem2):
        wid = lax.axis_index("s") * info.num_cores + lax.axis_index("c")
        base = wid * rpw
        pltpu.sync_copy(tok_hbm.at[pl.ds(base, rpw)], tid_v)
        pltpu.sync_copy(dst_hbm.at[pl.ds(base, rpw)], dst_v)
        pltpu.async_copy(emb_hbm.at[tid_v], rows_v, sem1).wait()
        pltpu.async_copy(rows_v, x_hbm.at[dst_v], sem2).wait()

    return gather_kernel(tokens, dst, emb)


def _gates1(x, wt, bsum):
    rows, k = x.shape
    blk = 512
    nblk = rows // blk

    def body(x_ref, w_ref, b_ref, gf_ref, gb_ref):
        xb = x_ref[...].astype(jnp.bfloat16)
        gf_ref[...] = (jnp.dot(xb, w_ref[0], preferred_element_type=jnp.float32)
                       + b_ref[0:1, :])
        gb_ref[...] = (jnp.dot(xb, w_ref[1], preferred_element_type=jnp.float32)
                       + b_ref[1:2, :])

    return pl.pallas_call(
        body,
        grid=(nblk,),
        in_specs=[
            pl.BlockSpec((blk, k), lambda i: (i, 0)),
            pl.BlockSpec((2, k, _G), lambda i: (0, 0, 0)),
            pl.BlockSpec((2, _G), lambda i: (0, 0)),
        ],
        out_specs=[
            pl.BlockSpec((blk, _G), lambda i: (i, 0)),
            pl.BlockSpec((blk, _G), lambda i: (i, 0)),
        ],
        out_shape=[
            jax.ShapeDtypeStruct((rows, _G), jnp.float32),
            jax.ShapeDtypeStruct((rows, _G), jnp.float32),
        ],
    )(x, wt, bsum)


def _gates2(xa, xb, wta, wtb, bsum):
    rows, k = xa.shape
    blk = 512
    nblk = rows // blk

    def body(xa_ref, xb_ref, wa_ref, wb_ref, b_ref, gf_ref, gb_ref):
        a = xa_ref[...].astype(jnp.bfloat16)
        b = xb_ref[...].astype(jnp.bfloat16)
        gf_ref[...] = (jnp.dot(a, wa_ref[0], preferred_element_type=jnp.float32)
                       + jnp.dot(b, wb_ref[0], preferred_element_type=jnp.float32)
                       + b_ref[0:1, :])
        gb_ref[...] = (jnp.dot(a, wa_ref[1], preferred_element_type=jnp.float32)
                       + jnp.dot(b, wb_ref[1], preferred_element_type=jnp.float32)
                       + b_ref[1:2, :])

    return pl.pallas_call(
        body,
        grid=(nblk,),
        in_specs=[
            pl.BlockSpec((blk, k), lambda i: (i, 0)),
            pl.BlockSpec((blk, k), lambda i: (i, 0)),
            pl.BlockSpec((2, k, _G), lambda i: (0, 0, 0)),
            pl.BlockSpec((2, k, _G), lambda i: (0, 0, 0)),
            pl.BlockSpec((2, _G), lambda i: (0, 0)),
        ],
        out_specs=[
            pl.BlockSpec((blk, _G), lambda i: (i, 0)),
            pl.BlockSpec((blk, _G), lambda i: (i, 0)),
        ],
        out_shape=[
            jax.ShapeDtypeStruct((rows, _G), jnp.float32),
            jax.ShapeDtypeStruct((rows, _G), jnp.float32),
        ],
    )(xa, xb, wta, wtb, bsum)


def _lstm_cell(g, h, c, m):
    i = jax.nn.sigmoid(g[:, 0:_H])
    f = jax.nn.sigmoid(g[:, _H:2 * _H])
    u = jnp.tanh(g[:, 2 * _H:3 * _H])
    o = jax.nn.sigmoid(g[:, 3 * _H:])
    cn = f * c + i * u
    hn = o * jnp.tanh(cn)
    sel = m > 0.5
    return jnp.where(sel, hn, h), jnp.where(sel, cn, c)


def _recur(gf, gb, mask, wf, wb, want_hs):
    rows = _CH * _B

    def body(gf_ref, gb_ref, mf_ref, mb_ref, wf_ref, wb_ref, *rest):
        outs = rest[:-4]
        hf, cf, hb, cb = rest[-4:]

        @pl.when(pl.program_id(0) == 0)
        def _init():
            hf[...] = jnp.zeros_like(hf)
            cf[...] = jnp.zeros_like(cf)
            hb[...] = jnp.zeros_like(hb)
            cb[...] = jnp.zeros_like(cb)

        def step(s, _):
            rf = pl.ds(s * _B, _B)
            g = gf_ref[rf, :] + jnp.dot(hf[...].astype(jnp.bfloat16),
                                        wf_ref[...],
                                        preferred_element_type=jnp.float32)
            hn, cn = _lstm_cell(g, hf[...], cf[...], mf_ref[rf, :])
            hf[...] = hn
            cf[...] = cn
            if want_hs:
                outs[0][rf, :] = hn

            rb = pl.ds(((_CH - 1) - s) * _B, _B)
            g2 = gb_ref[rb, :] + jnp.dot(hb[...].astype(jnp.bfloat16),
                                         wb_ref[...],
                                         preferred_element_type=jnp.float32)
            hn2, cn2 = _lstm_cell(g2, hb[...], cb[...], mb_ref[rb, :])
            hb[...] = hn2
            cb[...] = cn2
            if want_hs:
                outs[1][rb, :] = hn2
            return 0

        lax.fori_loop(0, _CH, step, 0, unroll=8)
        if not want_hs:
            outs[0][:, 0:_H] = hf[...]
            outs[0][:, _H:] = hb[...]

    fwd_map = lambda k: (k, 0)
    bwd_map = lambda k: (_NK - 1 - k, 0)
    in_specs = [
        pl.BlockSpec((rows, _G), fwd_map),
        pl.BlockSpec((rows, _G), bwd_map),
        pl.BlockSpec((rows, 1), fwd_map),
        pl.BlockSpec((rows, 1), bwd_map),
        pl.BlockSpec((_H, _G), lambda k: (0, 0)),
        pl.BlockSpec((_H, _G), lambda k: (0, 0)),
    ]
    wf = wf.astype(jnp.bfloat16)
    wb = wb.astype(jnp.bfloat16)
    if want_hs:
        out_specs = [pl.BlockSpec((rows, _H), fwd_map),
                     pl.BlockSpec((rows, _H), bwd_map)]
        out_shape = [jax.ShapeDtypeStruct((_T * _B, _H), jnp.float32),
                     jax.ShapeDtypeStruct((_T * _B, _H), jnp.float32)]
    else:
        out_specs = [pl.BlockSpec((_B, 2 * _H), lambda k: (0, 0))]
        out_shape = [jax.ShapeDtypeStruct((_B, 2 * _H), jnp.float32)]

    return pl.pallas_call(
        body,
        grid=(_NK,),
        in_specs=in_specs,
        out_specs=out_specs,
        out_shape=out_shape,
        scratch_shapes=[
            pltpu.VMEM((_B, _H), jnp.float32),
            pltpu.VMEM((_B, _H), jnp.float32),
            pltpu.VMEM((_B, _H), jnp.float32),
            pltpu.VMEM((_B, _H), jnp.float32),
        ],
    )(gf, gb, mask, mask, wf, wb)


def kernel(tokens, cu_seqlens, emb, Wih0, Whh0, bih0, bhh0,
           Wih1, Whh1, bih1, bhh1):
    cu = cu_seqlens.astype(jnp.int32)
    lengths = cu[1:] - cu[:-1]
    t = jnp.arange(_T, dtype=jnp.int32)
    mask = (t[:, None] < lengths[None, :]).astype(jnp.float32).reshape(-1, 1)

    p = jnp.arange(tokens.shape[0], dtype=jnp.int32)
    b = jnp.searchsorted(cu, p, side="right").astype(jnp.int32) - 1
    dst = (p - cu[b]) * _B + b

    x = _sc_gather(tokens.astype(jnp.int32), dst, emb)

    w0t = jnp.transpose(Wih0, (0, 2, 1)).astype(jnp.bfloat16)
    g0f, g0b = _gates1(x, w0t, bih0 + bhh0)
    whh0t = jnp.transpose(Whh0, (0, 2, 1))
    hsf, hsb = _recur(g0f, g0b, mask, whh0t[0], whh0t[1], want_hs=True)

    w1t = jnp.transpose(Wih1, (0, 2, 1)).astype(jnp.bfloat16)
    g1f, g1b = _gates2(hsf, hsb, w1t[:, :_H, :], w1t[:, _H:, :], bih1 + bhh1)
    whh1t = jnp.transpose(Whh1, (0, 2, 1))
    (h_final,) = _recur(g1f, g1b, mask, whh1t[0], whh1t[1], want_hs=False)
    return h_final
